# baseline (speedup 1.0000x reference)
import functools

import jax
import jax.numpy as jnp
from jax import lax
from jax.experimental import pallas as pl
from jax.experimental.pallas import tpu as pltpu
from jax.experimental.pallas import tpu_sc as plsc

N_ATOMS = 500000
SPECIES = 10
NSEG = 16
LANES = 16
NW = 32

AFLAT = N_ATOMS * SPECIES
PFLAT = N_ATOMS * 3
CA = 20000
CP = 12000
NGA = AFLAT // CA
NGP = PFLAT // CP
GPTA = -(-NGA // NW)
GPTP = -(-NGP // NW)


def _phase(src_hbm, bufs, sems, cut_ref, wvs, cdim, n_chunks, per_worker,
           cut_acc, acc_base, wid):
    groups = cdim // (5 * LANES)
    g_lo = wid * per_worker
    g_hi = jnp.minimum(g_lo + per_worker, n_chunks)
    cvec = cut_ref[...]
    cks = [cvec[k] for k in range(NSEG)]
    iota = lax.iota(jnp.int32, LANES)
    zero_v = jnp.zeros((LANES,), jnp.float32)

    def weighted(x, u):
        if wvs is None:
            return x * x
        return x * wvs[u]

    def issue(g, b):
        pltpu.async_copy(src_hbm.at[pl.ds(g * cdim, cdim)], bufs[b],
                         sems[b])

    def wait(b):
        pltpu.make_async_copy(src_hbm.at[pl.ds(0, cdim)], bufs[b],
                              sems[b]).wait()

    def chunk_total(bufb):
        @plsc.parallel_loop(0, cdim, step=5 * LANES, unroll=4,
                            carry=(zero_v,) * 5)
        def accs(base, accs):
            return tuple(accs[u] + weighted(bufb[pl.ds(base + u * LANES,
                                                       LANES)], u)
                         for u in range(5))
        return jnp.sum(accs[0] + accs[1] + accs[2] + accs[3] + accs[4])

    def masked_partial(bufb, lo, ck):
        @plsc.parallel_loop(0, cdim, step=5 * LANES, unroll=2,
                            carry=(zero_v,) * 5)
        def accs(base, accs):
            out = []
            for u in range(5):
                off = base + u * LANES
                x = weighted(bufb[pl.ds(off, LANES)], u)
                f = lo + off + iota
                out.append(accs[u] + jnp.where(f < ck, x, 0.0))
            return tuple(out)
        return jnp.sum(accs[0] + accs[1] + accs[2] + accs[3] + accs[4])

    for b in range(2):
        @pl.when(g_lo + b < g_hi)
        def _():
            issue(g_lo + b, b)

    def outer(t, carry):
        run, cutvec = carry
        for b in range(2):
            g = g_lo + 2 * t + b
            active = g < g_hi
            lo = g * cdim
            hi = lo + cdim

            @pl.when(active)
            def _():
                wait(b)
            bufb = bufs[b]
            total = chunk_total(bufb)

            for k in range(NSEG):
                def hit(bufb=bufb, lo=lo, k=k, run=run, cutvec=cutvec):
                    part = masked_partial(bufb, lo, cks[k])
                    return jnp.where(iota == k, run + part, cutvec)

                def miss(cutvec=cutvec):
                    return cutvec

                straddle = jnp.logical_and(
                    active,
                    jnp.logical_and(cks[k] >= lo, cks[k] < hi))
                cutvec = lax.cond(straddle, hit, miss)

            @pl.when(jnp.logical_and(active, g + 2 < g_hi))
            def _():
                issue(g + 2, b)
            run = jnp.where(active, run + total, run)
        return run, cutvec

    n_my = g_hi - g_lo
    run_total, cutvec = lax.fori_loop(
        0, (n_my + 1) // 2, outer,
        (jnp.float32(0.0), jnp.zeros((NSEG,), jnp.float32)))

    my_hi = g_hi * cdim
    past = cvec >= my_hi
    cutvec = jnp.where(past, run_total, cutvec)
    cut_acc[pl.ds(acc_base, NSEG)] = cutvec


def _sc_body(attrs_hbm, pos_hbm, c10_hbm, c3_hbm, wpat_hbm, out_hbm,
             abuf0, abuf1, pbuf0, pbuf1, cva, cvp, wv, cut_acc, sem0, sem1):
    wid = lax.axis_index("c") * 16 + lax.axis_index("s")
    pltpu.sync_copy(c10_hbm, cva)
    pltpu.sync_copy(c3_hbm, cvp)
    pltpu.sync_copy(wpat_hbm, wv)
    wvs = [wv[pl.ds(u * LANES, LANES)] for u in range(5)]
    sems = (sem0, sem1)
    _phase(attrs_hbm, (abuf0, abuf1), sems, cva, wvs, CA, NGA, GPTA,
           cut_acc, 0, wid)
    _phase(pos_hbm, (pbuf0, pbuf1), sems, cvp, None, CP, NGP, GPTP,
           cut_acc, NSEG, wid)
    pltpu.sync_copy(cut_acc, out_hbm.at[wid])


_mace_sc = functools.partial(
    pl.kernel,
    out_type=jax.ShapeDtypeStruct((NW, 2 * NSEG), jnp.float32),
    mesh=plsc.VectorSubcoreMesh(core_axis_name="c", subcore_axis_name="s"),
    scratch_types=[
        pltpu.VMEM((CA,), jnp.float32),
        pltpu.VMEM((CA,), jnp.float32),
        pltpu.VMEM((CP,), jnp.float32),
        pltpu.VMEM((CP,), jnp.float32),
        pltpu.VMEM((NSEG,), jnp.int32),
        pltpu.VMEM((NSEG,), jnp.int32),
        pltpu.VMEM((80,), jnp.float32),
        pltpu.VMEM((2 * NSEG,), jnp.float32),
        pltpu.SemaphoreType.DMA,
        pltpu.SemaphoreType.DMA,
    ],
    compiler_params=pltpu.CompilerParams(needs_layout_passes=False),
)(_sc_body)


def kernel(node_attrs, positions, ptr, W, b):
    attrs_flat = node_attrs.reshape(-1)
    pos_flat = positions.reshape(-1)
    ptr = ptr.astype(jnp.int32)
    ends = ptr[1:]
    c10 = ends * SPECIES
    c3 = ends * 3
    wpat = jnp.tile(W.reshape(-1), 8)

    partials = _mace_sc(attrs_flat, pos_flat, c10, c3, wpat)
    colsum = jnp.sum(partials, axis=0)
    cum_a = colsum[:NSEG]
    cum_p = colsum[NSEG:]
    z1 = jnp.zeros((1,), jnp.float32)
    seg_a = cum_a - jnp.concatenate([z1, cum_a[:-1]])
    seg_p = cum_p - jnp.concatenate([z1, cum_p[:-1]])
    counts = (ptr[1:] - ptr[:-1]).astype(jnp.float32)
    return seg_a + 0.5 * seg_p + b[0] * counts


# speedup vs baseline: 3.2808x; 3.2808x over previous
import functools

import jax
import jax.numpy as jnp
from jax import lax
from jax.experimental import pallas as pl
from jax.experimental.pallas import tpu as pltpu
from jax.experimental.pallas import tpu_sc as plsc

N_ATOMS = 500000
SPECIES = 10
NSEG = 16
LANES = 16
NW = 32
CB = 8192
CE = 2000
NGE = N_ATOMS // CE
GPTE = -(-NGE // NW)


def _tc_body(attrs_ref, pos_ref, w_ref, out_ref):
    e = jnp.sum(attrs_ref[...] * w_ref[...], axis=1)
    p = pos_ref[...]
    out_ref[...] = e + 0.5 * jnp.sum(p * p, axis=1)


def _per_atom(node_attrs, positions, w_row):
    grid = (-(-N_ATOMS // CB),)
    return pl.pallas_call(
        _tc_body,
        grid=grid,
        in_specs=[
            pl.BlockSpec((CB, SPECIES), lambda i: (i, 0)),
            pl.BlockSpec((CB, 3), lambda i: (i, 0)),
            pl.BlockSpec((1, SPECIES), lambda i: (0, 0)),
        ],
        out_specs=pl.BlockSpec((CB,), lambda i: (i,)),
        out_shape=jax.ShapeDtypeStruct((N_ATOMS,), jnp.float32),
    )(node_attrs, positions, w_row)


def _sc_body(e_hbm, cuts_hbm, out_hbm, buf0, buf1, cutm, acc_out, sem0, sem1):
    wid = lax.axis_index("c") * 16 + lax.axis_index("s")
    pltpu.sync_copy(cuts_hbm, cutm)
    bufs = (buf0, buf1)
    sems = (sem0, sem1)
    g_lo = wid * GPTE
    g_hi = jnp.minimum(g_lo + GPTE, NGE)
    cvec = cutm[...]
    cks = [cvec[k] for k in range(NSEG)]
    iota = lax.iota(jnp.int32, LANES)
    zero_v = jnp.zeros((LANES,), jnp.float32)

    def issue(g, b):
        pltpu.async_copy(e_hbm.at[pl.ds(g * CE, CE)], bufs[b], sems[b])

    def wait(b):
        pltpu.make_async_copy(e_hbm.at[pl.ds(0, CE)], bufs[b],
                              sems[b]).wait()

    def chunk_total(bufb):
        @plsc.parallel_loop(0, CE, step=5 * LANES, unroll=4,
                            carry=(zero_v,) * 5)
        def accs(base, accs):
            return tuple(accs[u] + bufb[pl.ds(base + u * LANES, LANES)]
                         for u in range(5))
        return jnp.sum(accs[0] + accs[1] + accs[2] + accs[3] + accs[4])

    def masked_partial(bufb, lo, ck):
        @plsc.parallel_loop(0, CE, step=5 * LANES, unroll=2,
                            carry=(zero_v,) * 5)
        def accs(base, accs):
            out = []
            for u in range(5):
                off = base + u * LANES
                x = bufb[pl.ds(off, LANES)]
                f = lo + off + iota
                out.append(accs[u] + jnp.where(f < ck, x, 0.0))
            return tuple(out)
        return jnp.sum(accs[0] + accs[1] + accs[2] + accs[3] + accs[4])

    for b in range(2):
        @pl.when(g_lo + b < g_hi)
        def _():
            issue(g_lo + b, b)

    def outer(t, carry):
        run, cutvec = carry
        for b in range(2):
            g = g_lo + 2 * t + b
            active = g < g_hi
            lo = g * CE
            hi = lo + CE

            @pl.when(active)
            def _():
                wait(b)
            bufb = bufs[b]
            total = chunk_total(bufb)

            for k in range(NSEG):
                def hit(bufb=bufb, lo=lo, k=k, run=run, cutvec=cutvec):
                    part = masked_partial(bufb, lo, cks[k])
                    return jnp.where(iota == k, run + part, cutvec)

                def miss(cutvec=cutvec):
                    return cutvec

                straddle = jnp.logical_and(
                    active,
                    jnp.logical_and(cks[k] >= lo, cks[k] < hi))
                cutvec = lax.cond(straddle, hit, miss)

            @pl.when(jnp.logical_and(active, g + 2 < g_hi))
            def _():
                issue(g + 2, b)
            run = jnp.where(active, run + total, run)
        return run, cutvec

    n_my = g_hi - g_lo
    run_total, cutvec = lax.fori_loop(
        0, (n_my + 1) // 2, outer,
        (jnp.float32(0.0), jnp.zeros((NSEG,), jnp.float32)))

    my_hi = g_hi * CE
    cutvec = jnp.where(cvec >= my_hi, run_total, cutvec)
    acc_out[...] = cutvec
    pltpu.sync_copy(acc_out, out_hbm.at[wid])


_seg_sum_sc = functools.partial(
    pl.kernel,
    out_type=jax.ShapeDtypeStruct((NW, NSEG), jnp.float32),
    mesh=plsc.VectorSubcoreMesh(core_axis_name="c", subcore_axis_name="s"),
    scratch_types=[
        pltpu.VMEM((CE,), jnp.float32),
        pltpu.VMEM((CE,), jnp.float32),
        pltpu.VMEM((NSEG,), jnp.int32),
        pltpu.VMEM((NSEG,), jnp.float32),
        pltpu.SemaphoreType.DMA,
        pltpu.SemaphoreType.DMA,
    ],
    compiler_params=pltpu.CompilerParams(needs_layout_passes=False),
)(_sc_body)


def kernel(node_attrs, positions, ptr, W, b):
    ptr = ptr.astype(jnp.int32)
    per_atom = _per_atom(node_attrs, positions, W)
    partials = _seg_sum_sc(per_atom, ptr[1:])
    cum = jnp.sum(partials, axis=0)
    z1 = jnp.zeros((1,), jnp.float32)
    seg = cum - jnp.concatenate([z1, cum[:-1]])
    counts = (ptr[1:] - ptr[:-1]).astype(jnp.float32)
    return seg + b[0] * counts


# speedup vs baseline: 4.4481x; 1.3558x over previous
import functools

import jax
import jax.numpy as jnp
from jax import lax
from jax.experimental import pallas as pl
from jax.experimental.pallas import tpu as pltpu
from jax.experimental.pallas import tpu_sc as plsc

N_ATOMS = 500000
SPECIES = 10
NSEG = 16
LANES = 16
NW = 32
CB = 8192
CE = 2000
NGE = N_ATOMS // CE
GPTE = -(-NGE // NW)


def _tc_body(attrs_ref, pos_ref, w_ref, h_ref, out_ref):
    eT = lax.dot_general(w_ref[...], attrs_ref[...],
                         (((1,), (1,)), ((), ())),
                         preferred_element_type=jnp.float32)
    p = pos_ref[...]
    pT = lax.dot_general(h_ref[...], p * p,
                         (((1,), (1,)), ((), ())),
                         preferred_element_type=jnp.float32)
    out_ref[...] = (eT + pT).reshape(CB)


def _per_atom(node_attrs, positions, w_row, h_row):
    grid = (-(-N_ATOMS // CB),)
    return pl.pallas_call(
        _tc_body,
        grid=grid,
        in_specs=[
            pl.BlockSpec((CB, SPECIES), lambda i: (i, 0)),
            pl.BlockSpec((CB, 3), lambda i: (i, 0)),
            pl.BlockSpec((1, SPECIES), lambda i: (0, 0)),
            pl.BlockSpec((1, 3), lambda i: (0, 0)),
        ],
        out_specs=pl.BlockSpec((CB,), lambda i: (i,)),
        out_shape=jax.ShapeDtypeStruct((N_ATOMS,), jnp.float32),
    )(node_attrs, positions, w_row, h_row)


def _sc_body(e_hbm, cuts_hbm, out_hbm, buf0, buf1, cutm, acc_out, sem0, sem1):
    wid = lax.axis_index("c") * 16 + lax.axis_index("s")
    pltpu.sync_copy(cuts_hbm, cutm)
    bufs = (buf0, buf1)
    sems = (sem0, sem1)
    g_lo = wid * GPTE
    g_hi = jnp.minimum(g_lo + GPTE, NGE)
    cvec = cutm[...]
    cks = [cvec[k] for k in range(NSEG)]
    iota = lax.iota(jnp.int32, LANES)
    zero_v = jnp.zeros((LANES,), jnp.float32)

    def issue(g, b):
        pltpu.async_copy(e_hbm.at[pl.ds(g * CE, CE)], bufs[b], sems[b])

    def wait(b):
        pltpu.make_async_copy(e_hbm.at[pl.ds(0, CE)], bufs[b],
                              sems[b]).wait()

    def chunk_total(bufb):
        @plsc.parallel_loop(0, CE, step=5 * LANES, unroll=4,
                            carry=(zero_v,) * 5)
        def accs(base, accs):
            return tuple(accs[u] + bufb[pl.ds(base + u * LANES, LANES)]
                         for u in range(5))
        return jnp.sum(accs[0] + accs[1] + accs[2] + accs[3] + accs[4])

    def masked_partial(bufb, lo, ck):
        @plsc.parallel_loop(0, CE, step=5 * LANES, unroll=2,
                            carry=(zero_v,) * 5)
        def accs(base, accs):
            out = []
            for u in range(5):
                off = base + u * LANES
                x = bufb[pl.ds(off, LANES)]
                f = lo + off + iota
                out.append(accs[u] + jnp.where(f < ck, x, 0.0))
            return tuple(out)
        return jnp.sum(accs[0] + accs[1] + accs[2] + accs[3] + accs[4])

    for b in range(2):
        @pl.when(g_lo + b < g_hi)
        def _():
            issue(g_lo + b, b)

    def outer(t, carry):
        run, cutvec = carry
        for b in range(2):
            g = g_lo + 2 * t + b
            active = g < g_hi
            lo = g * CE
            hi = lo + CE

            @pl.when(active)
            def _():
                wait(b)
            bufb = bufs[b]
            total = chunk_total(bufb)

            for k in range(NSEG):
                def hit(bufb=bufb, lo=lo, k=k, run=run, cutvec=cutvec):
                    part = masked_partial(bufb, lo, cks[k])
                    return jnp.where(iota == k, run + part, cutvec)

                def miss(cutvec=cutvec):
                    return cutvec

                straddle = jnp.logical_and(
                    active,
                    jnp.logical_and(cks[k] >= lo, cks[k] < hi))
                cutvec = lax.cond(straddle, hit, miss)

            @pl.when(jnp.logical_and(active, g + 2 < g_hi))
            def _():
                issue(g + 2, b)
            run = jnp.where(active, run + total, run)
        return run, cutvec

    n_my = g_hi - g_lo
    run_total, cutvec = lax.fori_loop(
        0, (n_my + 1) // 2, outer,
        (jnp.float32(0.0), jnp.zeros((NSEG,), jnp.float32)))

    my_hi = g_hi * CE
    cutvec = jnp.where(cvec >= my_hi, run_total, cutvec)
    acc_out[...] = cutvec
    pltpu.sync_copy(acc_out, out_hbm.at[wid])


_seg_sum_sc = functools.partial(
    pl.kernel,
    out_type=jax.ShapeDtypeStruct((NW, NSEG), jnp.float32),
    mesh=plsc.VectorSubcoreMesh(core_axis_name="c", subcore_axis_name="s"),
    scratch_types=[
        pltpu.VMEM((CE,), jnp.float32),
        pltpu.VMEM((CE,), jnp.float32),
        pltpu.VMEM((NSEG,), jnp.int32),
        pltpu.VMEM((NSEG,), jnp.float32),
        pltpu.SemaphoreType.DMA,
        pltpu.SemaphoreType.DMA,
    ],
    compiler_params=pltpu.CompilerParams(needs_layout_passes=False),
)(_sc_body)


def kernel(node_attrs, positions, ptr, W, b):
    ptr = ptr.astype(jnp.int32)
    h_row = jnp.full((1, 3), 0.5, jnp.float32)
    per_atom = _per_atom(node_attrs, positions, W, h_row)
    partials = _seg_sum_sc(per_atom, ptr[1:])
    cum = jnp.sum(partials, axis=0)
    z1 = jnp.zeros((1,), jnp.float32)
    seg = cum - jnp.concatenate([z1, cum[:-1]])
    counts = (ptr[1:] - ptr[:-1]).astype(jnp.float32)
    return seg + b[0] * counts


# speedup vs baseline: 7.5313x; 1.6931x over previous
import functools

import jax
import jax.numpy as jnp
from jax import lax
from jax.experimental import pallas as pl
from jax.experimental.pallas import tpu as pltpu
from jax.experimental.pallas import tpu_sc as plsc

N_ATOMS = 500000
SPECIES = 10
NSEG = 16
LANES = 16
NW = 32
CB = 8192
CE = 2000
NGE = N_ATOMS // CE
GPTE = -(-NGE // NW)


def _tc_body(attrs_ref, pos_ref, w_ref, out_ref):
    eT = lax.dot_general(w_ref[...], attrs_ref[...],
                         (((1,), (1,)), ((), ())),
                         preferred_element_type=jnp.float32)
    p = pos_ref[...]
    pT = jnp.sum(p * p, axis=0, keepdims=True)
    out_ref[...] = (eT + 0.5 * pT).reshape(CB)


def _per_atom(node_attrs, pos_t, w_row):
    grid = (-(-N_ATOMS // CB),)
    return pl.pallas_call(
        _tc_body,
        grid=grid,
        in_specs=[
            pl.BlockSpec((CB, SPECIES), lambda i: (i, 0)),
            pl.BlockSpec((3, CB), lambda i: (0, i)),
            pl.BlockSpec((1, SPECIES), lambda i: (0, 0)),
        ],
        out_specs=pl.BlockSpec((CB,), lambda i: (i,)),
        out_shape=jax.ShapeDtypeStruct((N_ATOMS,), jnp.float32),
    )(node_attrs, pos_t, w_row)


def _sc_body(e_hbm, cuts_hbm, out_hbm, buf0, buf1, cutm, acc_out, sem0, sem1):
    wid = lax.axis_index("c") * 16 + lax.axis_index("s")
    pltpu.sync_copy(cuts_hbm, cutm)
    bufs = (buf0, buf1)
    sems = (sem0, sem1)
    g_lo = wid * GPTE
    g_hi = jnp.minimum(g_lo + GPTE, NGE)
    cvec = cutm[...]
    cks = [cvec[k] for k in range(NSEG)]
    iota = lax.iota(jnp.int32, LANES)
    zero_v = jnp.zeros((LANES,), jnp.float32)

    def issue(g, b):
        pltpu.async_copy(e_hbm.at[pl.ds(g * CE, CE)], bufs[b], sems[b])

    def wait(b):
        pltpu.make_async_copy(e_hbm.at[pl.ds(0, CE)], bufs[b],
                              sems[b]).wait()

    def chunk_total(bufb):
        @plsc.parallel_loop(0, CE, step=5 * LANES, unroll=4,
                            carry=(zero_v,) * 5)
        def accs(base, accs):
            return tuple(accs[u] + bufb[pl.ds(base + u * LANES, LANES)]
                         for u in range(5))
        return jnp.sum(accs[0] + accs[1] + accs[2] + accs[3] + accs[4])

    def masked_partial(bufb, lo, ck):
        @plsc.parallel_loop(0, CE, step=5 * LANES, unroll=2,
                            carry=(zero_v,) * 5)
        def accs(base, accs):
            out = []
            for u in range(5):
                off = base + u * LANES
                x = bufb[pl.ds(off, LANES)]
                f = lo + off + iota
                out.append(accs[u] + jnp.where(f < ck, x, 0.0))
            return tuple(out)
        return jnp.sum(accs[0] + accs[1] + accs[2] + accs[3] + accs[4])

    for b in range(2):
        @pl.when(g_lo + b < g_hi)
        def _():
            issue(g_lo + b, b)

    def outer(t, carry):
        run, cutvec = carry
        for b in range(2):
            g = g_lo + 2 * t + b
            active = g < g_hi
            lo = g * CE
            hi = lo + CE

            @pl.when(active)
            def _():
                wait(b)
            bufb = bufs[b]
            total = chunk_total(bufb)

            for k in range(NSEG):
                def hit(bufb=bufb, lo=lo, k=k, run=run, cutvec=cutvec):
                    part = masked_partial(bufb, lo, cks[k])
                    return jnp.where(iota == k, run + part, cutvec)

                def miss(cutvec=cutvec):
                    return cutvec

                straddle = jnp.logical_and(
                    active,
                    jnp.logical_and(cks[k] >= lo, cks[k] < hi))
                cutvec = lax.cond(straddle, hit, miss)

            @pl.when(jnp.logical_and(active, g + 2 < g_hi))
            def _():
                issue(g + 2, b)
            run = jnp.where(active, run + total, run)
        return run, cutvec

    n_my = g_hi - g_lo
    run_total, cutvec = lax.fori_loop(
        0, (n_my + 1) // 2, outer,
        (jnp.float32(0.0), jnp.zeros((NSEG,), jnp.float32)))

    my_hi = g_hi * CE
    cutvec = jnp.where(cvec >= my_hi, run_total, cutvec)
    acc_out[...] = cutvec
    pltpu.sync_copy(acc_out, out_hbm.at[wid])


_seg_sum_sc = functools.partial(
    pl.kernel,
    out_type=jax.ShapeDtypeStruct((NW, NSEG), jnp.float32),
    mesh=plsc.VectorSubcoreMesh(core_axis_name="c", subcore_axis_name="s"),
    scratch_types=[
        pltpu.VMEM((CE,), jnp.float32),
        pltpu.VMEM((CE,), jnp.float32),
        pltpu.VMEM((NSEG,), jnp.int32),
        pltpu.VMEM((NSEG,), jnp.float32),
        pltpu.SemaphoreType.DMA,
        pltpu.SemaphoreType.DMA,
    ],
    compiler_params=pltpu.CompilerParams(needs_layout_passes=False),
)(_sc_body)


def kernel(node_attrs, positions, ptr, W, b):
    ptr = ptr.astype(jnp.int32)
    per_atom = _per_atom(node_attrs, positions.T, W)
    partials = _seg_sum_sc(per_atom, ptr[1:])
    cum = jnp.sum(partials, axis=0)
    z1 = jnp.zeros((1,), jnp.float32)
    seg = cum - jnp.concatenate([z1, cum[:-1]])
    counts = (ptr[1:] - ptr[:-1]).astype(jnp.float32)
    return seg + b[0] * counts


# speedup vs baseline: 24.1948x; 3.2126x over previous
import functools

import jax
import jax.numpy as jnp
from jax import lax
from jax.experimental import pallas as pl
from jax.experimental.pallas import tpu as pltpu
from jax.experimental.pallas import tpu_sc as plsc

N_ATOMS = 500000
SPECIES = 10
NSEG = 16
LANES = 16
NW = 32
CB = 8192
CE = 2000
NGE = N_ATOMS // CE
GPTE = -(-NGE // NW)


def _tc_body(attrs_ref, pos_ref, w_ref, out_ref):
    a = attrs_ref[...]
    eT = jnp.sum(a * w_ref[...], axis=0, keepdims=True)
    p = pos_ref[...]
    pT = jnp.sum(p * p, axis=0, keepdims=True)
    out_ref[...] = (eT + 0.5 * pT).reshape(CB)


def _per_atom(attrs_t, pos_t, w_col):
    grid = (-(-N_ATOMS // CB),)
    return pl.pallas_call(
        _tc_body,
        grid=grid,
        in_specs=[
            pl.BlockSpec((SPECIES, CB), lambda i: (0, i)),
            pl.BlockSpec((3, CB), lambda i: (0, i)),
            pl.BlockSpec((SPECIES, 1), lambda i: (0, 0)),
        ],
        out_specs=pl.BlockSpec((CB,), lambda i: (i,)),
        out_shape=jax.ShapeDtypeStruct((N_ATOMS,), jnp.float32),
    )(attrs_t, pos_t, w_col)


def _sc_body(e_hbm, cuts_hbm, out_hbm, buf0, buf1, cutm, acc_out, sem0, sem1):
    wid = lax.axis_index("c") * 16 + lax.axis_index("s")
    pltpu.sync_copy(cuts_hbm, cutm)
    bufs = (buf0, buf1)
    sems = (sem0, sem1)
    g_lo = wid * GPTE
    g_hi = jnp.minimum(g_lo + GPTE, NGE)
    cvec = cutm[...]
    cks = [cvec[k] for k in range(NSEG)]
    iota = lax.iota(jnp.int32, LANES)
    zero_v = jnp.zeros((LANES,), jnp.float32)

    def issue(g, b):
        pltpu.async_copy(e_hbm.at[pl.ds(g * CE, CE)], bufs[b], sems[b])

    def wait(b):
        pltpu.make_async_copy(e_hbm.at[pl.ds(0, CE)], bufs[b],
                              sems[b]).wait()

    def chunk_total(bufb):
        @plsc.parallel_loop(0, CE, step=5 * LANES, unroll=4,
                            carry=(zero_v,) * 5)
        def accs(base, accs):
            return tuple(accs[u] + bufb[pl.ds(base + u * LANES, LANES)]
                         for u in range(5))
        return jnp.sum(accs[0] + accs[1] + accs[2] + accs[3] + accs[4])

    def masked_partial(bufb, lo, ck):
        @plsc.parallel_loop(0, CE, step=5 * LANES, unroll=2,
                            carry=(zero_v,) * 5)
        def accs(base, accs):
            out = []
            for u in range(5):
                off = base + u * LANES
                x = bufb[pl.ds(off, LANES)]
                f = lo + off + iota
                out.append(accs[u] + jnp.where(f < ck, x, 0.0))
            return tuple(out)
        return jnp.sum(accs[0] + accs[1] + accs[2] + accs[3] + accs[4])

    for b in range(2):
        @pl.when(g_lo + b < g_hi)
        def _():
            issue(g_lo + b, b)

    def outer(t, carry):
        run, cutvec = carry
        for b in range(2):
            g = g_lo + 2 * t + b
            active = g < g_hi
            lo = g * CE
            hi = lo + CE

            @pl.when(active)
            def _():
                wait(b)
            bufb = bufs[b]
            total = chunk_total(bufb)

            for k in range(NSEG):
                def hit(bufb=bufb, lo=lo, k=k, run=run, cutvec=cutvec):
                    part = masked_partial(bufb, lo, cks[k])
                    return jnp.where(iota == k, run + part, cutvec)

                def miss(cutvec=cutvec):
                    return cutvec

                straddle = jnp.logical_and(
                    active,
                    jnp.logical_and(cks[k] >= lo, cks[k] < hi))
                cutvec = lax.cond(straddle, hit, miss)

            @pl.when(jnp.logical_and(active, g + 2 < g_hi))
            def _():
                issue(g + 2, b)
            run = jnp.where(active, run + total, run)
        return run, cutvec

    n_my = g_hi - g_lo
    run_total, cutvec = lax.fori_loop(
        0, (n_my + 1) // 2, outer,
        (jnp.float32(0.0), jnp.zeros((NSEG,), jnp.float32)))

    my_hi = g_hi * CE
    cutvec = jnp.where(cvec >= my_hi, run_total, cutvec)
    acc_out[...] = cutvec
    pltpu.sync_copy(acc_out, out_hbm.at[wid])


_seg_sum_sc = functools.partial(
    pl.kernel,
    out_type=jax.ShapeDtypeStruct((NW, NSEG), jnp.float32),
    mesh=plsc.VectorSubcoreMesh(core_axis_name="c", subcore_axis_name="s"),
    scratch_types=[
        pltpu.VMEM((CE,), jnp.float32),
        pltpu.VMEM((CE,), jnp.float32),
        pltpu.VMEM((NSEG,), jnp.int32),
        pltpu.VMEM((NSEG,), jnp.float32),
        pltpu.SemaphoreType.DMA,
        pltpu.SemaphoreType.DMA,
    ],
    compiler_params=pltpu.CompilerParams(needs_layout_passes=False),
)(_sc_body)


def kernel(node_attrs, positions, ptr, W, b):
    ptr = ptr.astype(jnp.int32)
    per_atom = _per_atom(node_attrs.T, positions.T, W.T)
    partials = _seg_sum_sc(per_atom, ptr[1:])
    cum = jnp.sum(partials, axis=0)
    z1 = jnp.zeros((1,), jnp.float32)
    seg = cum - jnp.concatenate([z1, cum[:-1]])
    counts = (ptr[1:] - ptr[:-1]).astype(jnp.float32)
    return seg + b[0] * counts


# speedup vs baseline: 33.0910x; 1.3677x over previous
import functools

import jax
import jax.numpy as jnp
from jax import lax
from jax.experimental import pallas as pl
from jax.experimental.pallas import tpu as pltpu
from jax.experimental.pallas import tpu_sc as plsc

N_ATOMS = 500000
SPECIES = 10
NSEG = 16
LANES = 16
NW = 32
CB = 32768
CE = 2000
NGE = N_ATOMS // CE
GPTE = -(-NGE // NW)


def _tc_body(attrs_ref, pos_ref, w_ref, out_ref):
    a = attrs_ref[...]
    eT = jnp.sum(a * w_ref[...], axis=0, keepdims=True)
    p = pos_ref[...]
    pT = jnp.sum(p * p, axis=0, keepdims=True)
    out_ref[...] = (eT + 0.5 * pT).reshape(CB)


def _per_atom(attrs_t, pos_t, w_col):
    grid = (-(-N_ATOMS // CB),)
    return pl.pallas_call(
        _tc_body,
        grid=grid,
        in_specs=[
            pl.BlockSpec((SPECIES, CB), lambda i: (0, i)),
            pl.BlockSpec((3, CB), lambda i: (0, i)),
            pl.BlockSpec((SPECIES, 1), lambda i: (0, 0)),
        ],
        out_specs=pl.BlockSpec((CB,), lambda i: (i,)),
        out_shape=jax.ShapeDtypeStruct((N_ATOMS,), jnp.float32),
    )(attrs_t, pos_t, w_col)


def _sc_body(e_hbm, cuts_hbm, out_hbm, buf0, buf1, cutm, acc_out, sem0, sem1):
    wid = lax.axis_index("c") * 16 + lax.axis_index("s")
    pltpu.sync_copy(cuts_hbm, cutm)
    bufs = (buf0, buf1)
    sems = (sem0, sem1)
    g_lo = wid * GPTE
    g_hi = jnp.minimum(g_lo + GPTE, NGE)
    cvec = cutm[...]
    cks = [cvec[k] for k in range(NSEG)]
    iota = lax.iota(jnp.int32, LANES)
    zero_v = jnp.zeros((LANES,), jnp.float32)

    def issue(g, b):
        pltpu.async_copy(e_hbm.at[pl.ds(g * CE, CE)], bufs[b], sems[b])

    def wait(b):
        pltpu.make_async_copy(e_hbm.at[pl.ds(0, CE)], bufs[b],
                              sems[b]).wait()

    def chunk_total(bufb):
        @plsc.parallel_loop(0, CE, step=5 * LANES, unroll=4,
                            carry=(zero_v,) * 5)
        def accs(base, accs):
            return tuple(accs[u] + bufb[pl.ds(base + u * LANES, LANES)]
                         for u in range(5))
        return jnp.sum(accs[0] + accs[1] + accs[2] + accs[3] + accs[4])

    def masked_partial(bufb, lo, ck):
        @plsc.parallel_loop(0, CE, step=5 * LANES, unroll=2,
                            carry=(zero_v,) * 5)
        def accs(base, accs):
            out = []
            for u in range(5):
                off = base + u * LANES
                x = bufb[pl.ds(off, LANES)]
                f = lo + off + iota
                out.append(accs[u] + jnp.where(f < ck, x, 0.0))
            return tuple(out)
        return jnp.sum(accs[0] + accs[1] + accs[2] + accs[3] + accs[4])

    for b in range(2):
        @pl.when(g_lo + b < g_hi)
        def _():
            issue(g_lo + b, b)

    def outer(t, carry):
        run, cutvec = carry
        for b in range(2):
            g = g_lo + 2 * t + b
            active = g < g_hi
            lo = g * CE
            hi = lo + CE

            @pl.when(active)
            def _():
                wait(b)
            bufb = bufs[b]
            total = chunk_total(bufb)

            for k in range(NSEG):
                def hit(bufb=bufb, lo=lo, k=k, run=run, cutvec=cutvec):
                    part = masked_partial(bufb, lo, cks[k])
                    return jnp.where(iota == k, run + part, cutvec)

                def miss(cutvec=cutvec):
                    return cutvec

                straddle = jnp.logical_and(
                    active,
                    jnp.logical_and(cks[k] >= lo, cks[k] < hi))
                cutvec = lax.cond(straddle, hit, miss)

            @pl.when(jnp.logical_and(active, g + 2 < g_hi))
            def _():
                issue(g + 2, b)
            run = jnp.where(active, run + total, run)
        return run, cutvec

    n_my = g_hi - g_lo
    run_total, cutvec = lax.fori_loop(
        0, (n_my + 1) // 2, outer,
        (jnp.float32(0.0), jnp.zeros((NSEG,), jnp.float32)))

    my_hi = g_hi * CE
    cutvec = jnp.where(cvec >= my_hi, run_total, cutvec)
    acc_out[...] = cutvec
    pltpu.sync_copy(acc_out, out_hbm.at[wid])


_seg_sum_sc = functools.partial(
    pl.kernel,
    out_type=jax.ShapeDtypeStruct((NW, NSEG), jnp.float32),
    mesh=plsc.VectorSubcoreMesh(core_axis_name="c", subcore_axis_name="s"),
    scratch_types=[
        pltpu.VMEM((CE,), jnp.float32),
        pltpu.VMEM((CE,), jnp.float32),
        pltpu.VMEM((NSEG,), jnp.int32),
        pltpu.VMEM((NSEG,), jnp.float32),
        pltpu.SemaphoreType.DMA,
        pltpu.SemaphoreType.DMA,
    ],
    compiler_params=pltpu.CompilerParams(needs_layout_passes=False),
)(_sc_body)


def kernel(node_attrs, positions, ptr, W, b):
    ptr = ptr.astype(jnp.int32)
    per_atom = _per_atom(node_attrs.T, positions.T, W.T)
    partials = _seg_sum_sc(per_atom, ptr[1:])
    cum = jnp.sum(partials, axis=0)
    z1 = jnp.zeros((1,), jnp.float32)
    seg = cum - jnp.concatenate([z1, cum[:-1]])
    counts = (ptr[1:] - ptr[:-1]).astype(jnp.float32)
    return seg + b[0] * counts


# speedup vs baseline: 37.9774x; 1.1477x over previous
import functools

import jax
import jax.numpy as jnp
from jax import lax
from jax.experimental import pallas as pl
from jax.experimental.pallas import tpu as pltpu
from jax.experimental.pallas import tpu_sc as plsc

N_ATOMS = 500000
SPECIES = 10
NSEG = 16
LANES = 16
NW = 32
CB = 65536
CE = 4000
NGE = N_ATOMS // CE
GPTE = -(-NGE // NW)


def _tc_body(attrs_ref, pos_ref, w_ref, out_ref):
    a = attrs_ref[...]
    eT = jnp.sum(a * w_ref[...], axis=0, keepdims=True)
    p = pos_ref[...]
    pT = jnp.sum(p * p, axis=0, keepdims=True)
    out_ref[...] = (eT + 0.5 * pT).reshape(CB)


def _per_atom(attrs_t, pos_t, w_col):
    grid = (-(-N_ATOMS // CB),)
    return pl.pallas_call(
        _tc_body,
        grid=grid,
        in_specs=[
            pl.BlockSpec((SPECIES, CB), lambda i: (0, i)),
            pl.BlockSpec((3, CB), lambda i: (0, i)),
            pl.BlockSpec((SPECIES, 1), lambda i: (0, 0)),
        ],
        out_specs=pl.BlockSpec((CB,), lambda i: (i,)),
        out_shape=jax.ShapeDtypeStruct((N_ATOMS,), jnp.float32),
    )(attrs_t, pos_t, w_col)


def _sc_body(e_hbm, cuts_hbm, out_hbm, buf0, buf1, cutm, acc_out, sem0, sem1):
    wid = lax.axis_index("c") * 16 + lax.axis_index("s")
    pltpu.sync_copy(cuts_hbm, cutm)
    bufs = (buf0, buf1)
    sems = (sem0, sem1)
    g_lo = wid * GPTE
    g_hi = jnp.minimum(g_lo + GPTE, NGE)
    cvec = cutm[...]
    cks = [cvec[k] for k in range(NSEG)]
    iota = lax.iota(jnp.int32, LANES)
    zero_v = jnp.zeros((LANES,), jnp.float32)

    def issue(g, b):
        pltpu.async_copy(e_hbm.at[pl.ds(g * CE, CE)], bufs[b], sems[b])

    def wait(b):
        pltpu.make_async_copy(e_hbm.at[pl.ds(0, CE)], bufs[b],
                              sems[b]).wait()

    def chunk_total(bufb):
        @plsc.parallel_loop(0, CE, step=5 * LANES, unroll=4,
                            carry=(zero_v,) * 5)
        def accs(base, accs):
            return tuple(accs[u] + bufb[pl.ds(base + u * LANES, LANES)]
                         for u in range(5))
        return jnp.sum(accs[0] + accs[1] + accs[2] + accs[3] + accs[4])

    def masked_partial(bufb, lo, ck):
        @plsc.parallel_loop(0, CE, step=5 * LANES, unroll=2,
                            carry=(zero_v,) * 5)
        def accs(base, accs):
            out = []
            for u in range(5):
                off = base + u * LANES
                x = bufb[pl.ds(off, LANES)]
                f = lo + off + iota
                out.append(accs[u] + jnp.where(f < ck, x, 0.0))
            return tuple(out)
        return jnp.sum(accs[0] + accs[1] + accs[2] + accs[3] + accs[4])

    for b in range(2):
        @pl.when(g_lo + b < g_hi)
        def _():
            issue(g_lo + b, b)

    def outer(t, carry):
        run, cutvec = carry
        for b in range(2):
            g = g_lo + 2 * t + b
            active = g < g_hi
            lo = g * CE
            hi = lo + CE

            @pl.when(active)
            def _():
                wait(b)
            bufb = bufs[b]
            total = chunk_total(bufb)

            for k in range(NSEG):
                def hit(bufb=bufb, lo=lo, k=k, run=run, cutvec=cutvec):
                    part = masked_partial(bufb, lo, cks[k])
                    return jnp.where(iota == k, run + part, cutvec)

                def miss(cutvec=cutvec):
                    return cutvec

                straddle = jnp.logical_and(
                    active,
                    jnp.logical_and(cks[k] >= lo, cks[k] < hi))
                cutvec = lax.cond(straddle, hit, miss)

            @pl.when(jnp.logical_and(active, g + 2 < g_hi))
            def _():
                issue(g + 2, b)
            run = jnp.where(active, run + total, run)
        return run, cutvec

    n_my = g_hi - g_lo
    run_total, cutvec = lax.fori_loop(
        0, (n_my + 1) // 2, outer,
        (jnp.float32(0.0), jnp.zeros((NSEG,), jnp.float32)))

    my_hi = g_hi * CE
    cutvec = jnp.where(cvec >= my_hi, run_total, cutvec)
    acc_out[...] = cutvec
    pltpu.sync_copy(acc_out, out_hbm.at[wid])


_seg_sum_sc = functools.partial(
    pl.kernel,
    out_type=jax.ShapeDtypeStruct((NW, NSEG), jnp.float32),
    mesh=plsc.VectorSubcoreMesh(core_axis_name="c", subcore_axis_name="s"),
    scratch_types=[
        pltpu.VMEM((CE,), jnp.float32),
        pltpu.VMEM((CE,), jnp.float32),
        pltpu.VMEM((NSEG,), jnp.int32),
        pltpu.VMEM((NSEG,), jnp.float32),
        pltpu.SemaphoreType.DMA,
        pltpu.SemaphoreType.DMA,
    ],
    compiler_params=pltpu.CompilerParams(needs_layout_passes=False),
)(_sc_body)


def kernel(node_attrs, positions, ptr, W, b):
    ptr = ptr.astype(jnp.int32)
    per_atom = _per_atom(node_attrs.T, positions.T, W.T)
    partials = _seg_sum_sc(per_atom, ptr[1:])
    cum = jnp.sum(partials, axis=0)
    z1 = jnp.zeros((1,), jnp.float32)
    seg = cum - jnp.concatenate([z1, cum[:-1]])
    counts = (ptr[1:] - ptr[:-1]).astype(jnp.float32)
    return seg + b[0] * counts


# speedup vs baseline: 38.3142x; 1.0089x over previous
import functools

import jax
import jax.numpy as jnp
from jax import lax
from jax.experimental import pallas as pl
from jax.experimental.pallas import tpu as pltpu
from jax.experimental.pallas import tpu_sc as plsc

N_ATOMS = 500000
SPECIES = 10
NSEG = 16
LANES = 16
NW = 32
CB = 131072
CE = 4000
NGE = N_ATOMS // CE
GPTE = -(-NGE // NW)


def _tc_body(attrs_ref, pos_ref, w_ref, out_ref):
    a = attrs_ref[...]
    eT = jnp.sum(a * w_ref[...], axis=0, keepdims=True)
    p = pos_ref[...]
    pT = jnp.sum(p * p, axis=0, keepdims=True)
    out_ref[...] = (eT + 0.5 * pT).reshape(CB)


def _per_atom(attrs_t, pos_t, w_col):
    grid = (-(-N_ATOMS // CB),)
    return pl.pallas_call(
        _tc_body,
        grid=grid,
        in_specs=[
            pl.BlockSpec((SPECIES, CB), lambda i: (0, i)),
            pl.BlockSpec((3, CB), lambda i: (0, i)),
            pl.BlockSpec((SPECIES, 1), lambda i: (0, 0)),
        ],
        out_specs=pl.BlockSpec((CB,), lambda i: (i,)),
        out_shape=jax.ShapeDtypeStruct((N_ATOMS,), jnp.float32),
    )(attrs_t, pos_t, w_col)


def _sc_body(e_hbm, cuts_hbm, out_hbm, buf0, buf1, cutm, acc_out, sem0, sem1):
    wid = lax.axis_index("c") * 16 + lax.axis_index("s")
    pltpu.sync_copy(cuts_hbm, cutm)
    bufs = (buf0, buf1)
    sems = (sem0, sem1)
    g_lo = wid * GPTE
    g_hi = jnp.minimum(g_lo + GPTE, NGE)
    cvec = cutm[...]
    cks = [cvec[k] for k in range(NSEG)]
    iota = lax.iota(jnp.int32, LANES)
    zero_v = jnp.zeros((LANES,), jnp.float32)

    def issue(g, b):
        pltpu.async_copy(e_hbm.at[pl.ds(g * CE, CE)], bufs[b], sems[b])

    def wait(b):
        pltpu.make_async_copy(e_hbm.at[pl.ds(0, CE)], bufs[b],
                              sems[b]).wait()

    def chunk_total(bufb):
        @plsc.parallel_loop(0, CE, step=5 * LANES, unroll=4,
                            carry=(zero_v,) * 5)
        def accs(base, accs):
            return tuple(accs[u] + bufb[pl.ds(base + u * LANES, LANES)]
                         for u in range(5))
        return jnp.sum(accs[0] + accs[1] + accs[2] + accs[3] + accs[4])

    def masked_partial(bufb, lo, ck):
        @plsc.parallel_loop(0, CE, step=5 * LANES, unroll=2,
                            carry=(zero_v,) * 5)
        def accs(base, accs):
            out = []
            for u in range(5):
                off = base + u * LANES
                x = bufb[pl.ds(off, LANES)]
                f = lo + off + iota
                out.append(accs[u] + jnp.where(f < ck, x, 0.0))
            return tuple(out)
        return jnp.sum(accs[0] + accs[1] + accs[2] + accs[3] + accs[4])

    for b in range(2):
        @pl.when(g_lo + b < g_hi)
        def _():
            issue(g_lo + b, b)

    def outer(t, carry):
        run, cutvec = carry
        for b in range(2):
            g = g_lo + 2 * t + b
            active = g < g_hi
            lo = g * CE
            hi = lo + CE

            @pl.when(active)
            def _():
                wait(b)
            bufb = bufs[b]
            total = chunk_total(bufb)

            for k in range(NSEG):
                def hit(bufb=bufb, lo=lo, k=k, run=run, cutvec=cutvec):
                    part = masked_partial(bufb, lo, cks[k])
                    return jnp.where(iota == k, run + part, cutvec)

                def miss(cutvec=cutvec):
                    return cutvec

                straddle = jnp.logical_and(
                    active,
                    jnp.logical_and(cks[k] >= lo, cks[k] < hi))
                cutvec = lax.cond(straddle, hit, miss)

            @pl.when(jnp.logical_and(active, g + 2 < g_hi))
            def _():
                issue(g + 2, b)
            run = jnp.where(active, run + total, run)
        return run, cutvec

    n_my = g_hi - g_lo
    run_total, cutvec = lax.fori_loop(
        0, (n_my + 1) // 2, outer,
        (jnp.float32(0.0), jnp.zeros((NSEG,), jnp.float32)))

    my_hi = g_hi * CE
    cutvec = jnp.where(cvec >= my_hi, run_total, cutvec)
    acc_out[...] = cutvec
    pltpu.sync_copy(acc_out, out_hbm.at[wid])


_seg_sum_sc = functools.partial(
    pl.kernel,
    out_type=jax.ShapeDtypeStruct((NW, NSEG), jnp.float32),
    mesh=plsc.VectorSubcoreMesh(core_axis_name="c", subcore_axis_name="s"),
    scratch_types=[
        pltpu.VMEM((CE,), jnp.float32),
        pltpu.VMEM((CE,), jnp.float32),
        pltpu.VMEM((NSEG,), jnp.int32),
        pltpu.VMEM((NSEG,), jnp.float32),
        pltpu.SemaphoreType.DMA,
        pltpu.SemaphoreType.DMA,
    ],
    compiler_params=pltpu.CompilerParams(needs_layout_passes=False),
)(_sc_body)


def kernel(node_attrs, positions, ptr, W, b):
    ptr = ptr.astype(jnp.int32)
    per_atom = _per_atom(node_attrs.T, positions.T, W.T)
    partials = _seg_sum_sc(per_atom, ptr[1:])
    cum = jnp.sum(partials, axis=0)
    z1 = jnp.zeros((1,), jnp.float32)
    seg = cum - jnp.concatenate([z1, cum[:-1]])
    counts = (ptr[1:] - ptr[:-1]).astype(jnp.float32)
    return seg + b[0] * counts
